# baseline (speedup 1.0000x reference)
import jax
import jax.numpy as jnp
from jax import lax
from jax.experimental import pallas as pl
from jax.experimental.pallas import tpu as pltpu
from jax.experimental.pallas import tpu_sc as plsc

D = 64
NUM_ROWS = 1000000
TOTAL = 16384 * 50


_TBLK = 8192
_NBLKS = 123
_Q = _TBLK // 4
_UROWS = _NBLKS * _Q


def _transform_body(t_ref, w_ref, b_ref, o_ref):
    def dot_cols(sl, wrows):
        r = lax.dot_general(
            t_ref[:, sl], w_ref[wrows, :],
            (((0,), (1,)), ((), ())),
            preferred_element_type=jnp.float32,
        )
        return r

    def pack(sl):
        lo = dot_cols(sl, slice(0, D // 2)) + b_ref[0:1, :]
        hi = dot_cols(sl, slice(D // 2, D)) + b_ref[1:2, :]
        lo16 = lax.bitcast_convert_type(
            lo.astype(jnp.bfloat16), jnp.uint16
        ).astype(jnp.uint32)
        hi16 = lax.bitcast_convert_type(
            hi.astype(jnp.bfloat16), jnp.uint16
        ).astype(jnp.uint32)
        return lo16 | (hi16 << 16)

    quarters = [pack(pl.ds(k * _Q, _Q)) for k in range(4)]
    o_ref[...] = lax.bitcast_convert_type(
        jnp.concatenate(quarters, axis=1), jnp.float32
    )


def _transform(table_t, W, b2):
    return pl.pallas_call(
        _transform_body,
        grid=(_NBLKS,),
        in_specs=[
            pl.BlockSpec((D, _TBLK), lambda i: (0, i)),
            pl.BlockSpec((D, D), lambda i: (0, 0)),
            pl.BlockSpec((2, D // 2), lambda i: (0, 0)),
        ],
        out_specs=pl.BlockSpec((_Q, 2 * D), lambda i: (i, 0)),
        out_shape=jax.ShapeDtypeStruct((_UROWS, 2 * D), jnp.float32),
    )(table_t, W, b2)


_DW = D // 2
_C = 128
_NBUF = 4
_NSPLIT = 2
_info = plsc.get_sparse_core_info()
_NC, _NS = _info.num_cores, _info.num_subcores
_NW = _NC * _NS
_PART = TOTAL // _NSPLIT
_PER_W = _PART // _NW
_CHUNKS = _PER_W // _C
_ITERS = _CHUNKS // _NBUF


def _gather_body(table_hbm, idx_hbm, out_hbm, idx_v, bufs, gsems):
    wid = lax.axis_index("s") * _NC + lax.axis_index("c")
    chunk0 = wid * _CHUNKS
    row0 = wid * _PER_W

    pltpu.sync_copy(idx_hbm.at[pl.ds(chunk0, _CHUNKS)], idx_v)

    def start_gather(j, b):
        pltpu.async_copy(table_hbm.at[idx_v.at[j]], bufs.at[b], gsems.at[b])

    def wait_gather(j, b):
        pltpu.make_async_copy(
            table_hbm.at[idx_v.at[j]], bufs.at[b], gsems.at[b]
        ).wait()

    for b in range(_NBUF):
        start_gather(b, b)

    def body(i, carry):
        for b in range(_NBUF):
            j = i * _NBUF + b
            wait_gather(j, b)
            pltpu.sync_copy(bufs.at[b], out_hbm.at[pl.ds(row0 + j * _C, _C)])

            @pl.when(i < _ITERS - 1)
            def _():
                start_gather(j + _NBUF, b)

        return carry

    lax.fori_loop(0, _ITERS, body, 0)


def _gather(table2, x2d):
    mesh = plsc.VectorSubcoreMesh(core_axis_name="c", subcore_axis_name="s")
    kfn = pl.kernel(
        _gather_body,
        out_type=jax.ShapeDtypeStruct((_PART, _DW), jnp.float32),
        mesh=mesh,
        scratch_types=[
            pltpu.VMEM((_CHUNKS, _C), jnp.int32),
            pltpu.VMEM((_NBUF, _C, _DW), jnp.float32),
            pltpu.SemaphoreType.DMA((_NBUF,)),
        ],
        compiler_params=pltpu.CompilerParams(use_tc_tiling_on_sc=False),
    )
    return kfn(table2, x2d)


def _assemble_body(g_ref, a_ref):
    w = lax.bitcast_convert_type(g_ref[0], jnp.uint32)
    lo = lax.bitcast_convert_type(
        (w & 0xFFFF).astype(jnp.uint16), jnp.bfloat16
    ).astype(jnp.float32)
    hi = lax.bitcast_convert_type(
        (w >> 16).astype(jnp.uint16), jnp.bfloat16
    ).astype(jnp.float32)
    loT = jnp.transpose(lo, (1, 0))
    hiT = jnp.transpose(hi, (1, 0))
    for m in range(4):
        a_ref[0, 0:D // 2, m * 4096:(m + 1) * 4096] = loT[32 * m:32 * m + 32, :]
        a_ref[0, D // 2:D, m * 4096:(m + 1) * 4096] = hiT[32 * m:32 * m + 32, :]


def _assemble_first(g3d, hist, batch, lsub):
    return pl.pallas_call(
        _assemble_body,
        grid=(lsub,),
        in_specs=[pl.BlockSpec((1, batch // 4, 2 * D), lambda l: (l, 0, 0))],
        out_specs=pl.BlockSpec((1, D, batch), lambda l: (l, 0, 0)),
        out_shape=jax.ShapeDtypeStruct((hist, D, batch), jnp.float32),
    )(g3d)


def _assemble_rest(g3d, acc, hist, batch, l0):
    def body(g_ref, _, a_ref):
        _assemble_body(g_ref, a_ref)

    lsub = hist - l0
    return pl.pallas_call(
        body,
        grid=(lsub,),
        in_specs=[
            pl.BlockSpec((1, batch // 4, 2 * D), lambda l: (l, 0, 0)),
            pl.BlockSpec(memory_space=pl.ANY),
        ],
        out_specs=pl.BlockSpec((1, D, batch), lambda l: (l + l0, 0, 0)),
        out_shape=jax.ShapeDtypeStruct((hist, D, batch), jnp.float32),
        input_output_aliases={1: 0},
    )(g3d, acc)


def kernel(x, table, W, b):
    batch, hist = x.shape
    u = _transform(table.T, W, b.reshape(2, D // 2))

    xi = x.astype(jnp.int32).T.reshape(hist, 4, batch // 4)
    xi = jnp.transpose(xi, (0, 2, 1)).reshape(-1)
    idx = (
        (xi - jnp.bitwise_and(xi, 8191))
        + (jnp.bitwise_and(xi, 2047) << 2)
        + (jnp.bitwise_and(xi, 8191) >> 11)
    )
    x2d = idx.reshape(TOTAL // _C, _C)

    uw = u.reshape(4 * _UROWS, _DW)
    hsub = hist // _NSPLIT
    rows = _PART // _C
    g1 = _gather(uw, x2d[0:rows])
    x2d_2, _ = lax.optimization_barrier((x2d[rows:2 * rows], g1))
    g2 = _gather(uw, x2d_2)
    a1 = _assemble_first(
        g1.reshape(hsub, batch // 4, 2 * D), hist, batch, hsub
    )
    a = _assemble_rest(
        g2.reshape(hsub, batch // 4, 2 * D), a1, hist, batch, hsub
    )
    return jnp.transpose(a, (2, 0, 1))


# speedup vs baseline: 1.0225x; 1.0225x over previous
import jax
import jax.numpy as jnp
from jax import lax
from jax.experimental import pallas as pl
from jax.experimental.pallas import tpu as pltpu
from jax.experimental.pallas import tpu_sc as plsc

D = 64
NUM_ROWS = 1000000
TOTAL = 16384 * 50


_TBLK = 16384
_NBLKS = 62
_Q = _TBLK // 4
_UROWS = _NBLKS * _Q


def _transform_body(t_ref, w_ref, b_ref, o_ref):
    def dot_cols(sl, wrows):
        r = lax.dot_general(
            t_ref[:, sl], w_ref[wrows, :],
            (((0,), (1,)), ((), ())),
            preferred_element_type=jnp.float32,
        )
        return r

    def pack(sl):
        lo = dot_cols(sl, slice(0, D // 2)) + b_ref[0:1, :]
        hi = dot_cols(sl, slice(D // 2, D)) + b_ref[1:2, :]
        lo16 = lax.bitcast_convert_type(
            lo.astype(jnp.bfloat16), jnp.uint16
        ).astype(jnp.uint32)
        hi16 = lax.bitcast_convert_type(
            hi.astype(jnp.bfloat16), jnp.uint16
        ).astype(jnp.uint32)
        return lo16 | (hi16 << 16)

    quarters = [pack(pl.ds(k * _Q, _Q)) for k in range(4)]
    o_ref[...] = lax.bitcast_convert_type(
        jnp.concatenate(quarters, axis=1), jnp.float32
    )


def _transform(table_t, W, b2):
    return pl.pallas_call(
        _transform_body,
        grid=(_NBLKS,),
        in_specs=[
            pl.BlockSpec((D, _TBLK), lambda i: (0, i)),
            pl.BlockSpec((D, D), lambda i: (0, 0)),
            pl.BlockSpec((2, D // 2), lambda i: (0, 0)),
        ],
        out_specs=pl.BlockSpec((_Q, 2 * D), lambda i: (i, 0)),
        out_shape=jax.ShapeDtypeStruct((_UROWS, 2 * D), jnp.float32),
    )(table_t, W, b2)


_DW = D // 2
_C = 128
_NBUF = 4
_NSPLIT = 2
_info = plsc.get_sparse_core_info()
_NC, _NS = _info.num_cores, _info.num_subcores
_NW = _NC * _NS
_PART = TOTAL // _NSPLIT
_PER_W = _PART // _NW
_CHUNKS = _PER_W // _C
_ITERS = _CHUNKS // _NBUF


def _gather_body(table_hbm, idx_hbm, out_hbm, idx_v, bufs, gsems):
    wid = lax.axis_index("s") * _NC + lax.axis_index("c")
    chunk0 = wid * _CHUNKS
    row0 = wid * _PER_W

    pltpu.sync_copy(idx_hbm.at[pl.ds(chunk0, _CHUNKS)], idx_v)

    def start_gather(j, b):
        pltpu.async_copy(table_hbm.at[idx_v.at[j]], bufs.at[b], gsems.at[b])

    def wait_gather(j, b):
        pltpu.make_async_copy(
            table_hbm.at[idx_v.at[j]], bufs.at[b], gsems.at[b]
        ).wait()

    for b in range(_NBUF):
        start_gather(b, b)

    def body(i, carry):
        for b in range(_NBUF):
            j = i * _NBUF + b
            wait_gather(j, b)
            pltpu.sync_copy(bufs.at[b], out_hbm.at[pl.ds(row0 + j * _C, _C)])

            @pl.when(i < _ITERS - 1)
            def _():
                start_gather(j + _NBUF, b)

        return carry

    lax.fori_loop(0, _ITERS, body, 0)


def _gather(table2, x2d):
    mesh = plsc.VectorSubcoreMesh(core_axis_name="c", subcore_axis_name="s")
    kfn = pl.kernel(
        _gather_body,
        out_type=jax.ShapeDtypeStruct((_PART, _DW), jnp.float32),
        mesh=mesh,
        scratch_types=[
            pltpu.VMEM((_CHUNKS, _C), jnp.int32),
            pltpu.VMEM((_NBUF, _C, _DW), jnp.float32),
            pltpu.SemaphoreType.DMA((_NBUF,)),
        ],
        compiler_params=pltpu.CompilerParams(use_tc_tiling_on_sc=False),
    )
    return kfn(table2, x2d)


def _assemble_body(g_ref, a_ref):
    w = lax.bitcast_convert_type(g_ref[0], jnp.uint32)
    lo = lax.bitcast_convert_type(
        (w & 0xFFFF).astype(jnp.uint16), jnp.bfloat16
    ).astype(jnp.float32)
    hi = lax.bitcast_convert_type(
        (w >> 16).astype(jnp.uint16), jnp.bfloat16
    ).astype(jnp.float32)
    loT = jnp.transpose(lo, (1, 0))
    hiT = jnp.transpose(hi, (1, 0))
    for m in range(4):
        a_ref[0, 0:D // 2, m * 4096:(m + 1) * 4096] = loT[32 * m:32 * m + 32, :]
        a_ref[0, D // 2:D, m * 4096:(m + 1) * 4096] = hiT[32 * m:32 * m + 32, :]


def _assemble_first(g3d, hist, batch, lsub):
    return pl.pallas_call(
        _assemble_body,
        grid=(lsub,),
        in_specs=[pl.BlockSpec((1, batch // 4, 2 * D), lambda l: (l, 0, 0))],
        out_specs=pl.BlockSpec((1, D, batch), lambda l: (l, 0, 0)),
        out_shape=jax.ShapeDtypeStruct((hist, D, batch), jnp.float32),
    )(g3d)


def _assemble_rest(g3d, acc, hist, batch, l0):
    def body(g_ref, _, a_ref):
        _assemble_body(g_ref, a_ref)

    lsub = hist - l0
    return pl.pallas_call(
        body,
        grid=(lsub,),
        in_specs=[
            pl.BlockSpec((1, batch // 4, 2 * D), lambda l: (l, 0, 0)),
            pl.BlockSpec(memory_space=pl.ANY),
        ],
        out_specs=pl.BlockSpec((1, D, batch), lambda l: (l + l0, 0, 0)),
        out_shape=jax.ShapeDtypeStruct((hist, D, batch), jnp.float32),
        input_output_aliases={1: 0},
    )(g3d, acc)


def kernel(x, table, W, b):
    batch, hist = x.shape
    u = _transform(table.T, W, b.reshape(2, D // 2))

    xi = x.astype(jnp.int32).T.reshape(hist, 4, batch // 4)
    xi = jnp.transpose(xi, (0, 2, 1)).reshape(-1)
    qbits = _Q.bit_length() - 1
    idx = (
        (xi - jnp.bitwise_and(xi, _TBLK - 1))
        + (jnp.bitwise_and(xi, _Q - 1) << 2)
        + (jnp.bitwise_and(xi, _TBLK - 1) >> qbits)
    )
    x2d = idx.reshape(TOTAL // _C, _C)

    uw = u.reshape(4 * _UROWS, _DW)
    hsub = hist // _NSPLIT
    rows = _PART // _C
    g1 = _gather(uw, x2d[0:rows])
    x2d_2, _ = lax.optimization_barrier((x2d[rows:2 * rows], g1))
    g2 = _gather(uw, x2d_2)
    a1 = _assemble_first(
        g1.reshape(hsub, batch // 4, 2 * D), hist, batch, hsub
    )
    a = _assemble_rest(
        g2.reshape(hsub, batch // 4, 2 * D), a1, hist, batch, hsub
    )
    return jnp.transpose(a, (2, 0, 1))


# speedup vs baseline: 1.0266x; 1.0041x over previous
import jax
import jax.numpy as jnp
from jax import lax
from jax.experimental import pallas as pl
from jax.experimental.pallas import tpu as pltpu
from jax.experimental.pallas import tpu_sc as plsc

D = 64
NUM_ROWS = 1000000
TOTAL = 16384 * 50


_TBLK = 16384
_NBLKS = 62
_Q = _TBLK // 4
_UROWS = _NBLKS * _Q


def _transform_body(t_ref, w_ref, b_ref, o_ref):
    def dot_cols(sl, wrows):
        r = lax.dot_general(
            t_ref[:, sl], w_ref[wrows, :],
            (((0,), (1,)), ((), ())),
            preferred_element_type=jnp.float32,
        )
        return r

    def pack(sl):
        lo = dot_cols(sl, slice(0, D // 2)) + b_ref[0:1, :]
        hi = dot_cols(sl, slice(D // 2, D)) + b_ref[1:2, :]
        lo16 = lax.bitcast_convert_type(
            lo.astype(jnp.bfloat16), jnp.uint16
        ).astype(jnp.uint32)
        hi16 = lax.bitcast_convert_type(
            hi.astype(jnp.bfloat16), jnp.uint16
        ).astype(jnp.uint32)
        return lo16 | (hi16 << 16)

    quarters = [pack(pl.ds(k * _Q, _Q)) for k in range(4)]
    o_ref[...] = lax.bitcast_convert_type(
        jnp.concatenate(quarters, axis=1), jnp.float32
    )


def _transform(table_t, W, b2):
    return pl.pallas_call(
        _transform_body,
        grid=(_NBLKS,),
        in_specs=[
            pl.BlockSpec((D, _TBLK), lambda i: (0, i)),
            pl.BlockSpec((D, D), lambda i: (0, 0)),
            pl.BlockSpec((2, D // 2), lambda i: (0, 0)),
        ],
        out_specs=pl.BlockSpec((_Q, 2 * D), lambda i: (i, 0)),
        out_shape=jax.ShapeDtypeStruct((_UROWS, 2 * D), jnp.float32),
    )(table_t, W, b2)


_DW = D // 2
_C = 128
_NBUF = 4
_NSPLIT = 1
_info = plsc.get_sparse_core_info()
_NC, _NS = _info.num_cores, _info.num_subcores
_NW = _NC * _NS
_PART = TOTAL // _NSPLIT
_PER_W = _PART // _NW
_CHUNKS = _PER_W // _C
_ITERS = _CHUNKS // _NBUF


def _gather_body(table_hbm, idx_hbm, out_hbm, idx_v, bufs, gsems):
    wid = lax.axis_index("s") * _NC + lax.axis_index("c")
    chunk0 = wid * _CHUNKS
    row0 = wid * _PER_W

    pltpu.sync_copy(idx_hbm.at[pl.ds(chunk0, _CHUNKS)], idx_v)

    def start_gather(j, b):
        pltpu.async_copy(table_hbm.at[idx_v.at[j]], bufs.at[b], gsems.at[b])

    def wait_gather(j, b):
        pltpu.make_async_copy(
            table_hbm.at[idx_v.at[j]], bufs.at[b], gsems.at[b]
        ).wait()

    for b in range(_NBUF):
        start_gather(b, b)

    def body(i, carry):
        for b in range(_NBUF):
            j = i * _NBUF + b
            wait_gather(j, b)
            pltpu.sync_copy(bufs.at[b], out_hbm.at[pl.ds(row0 + j * _C, _C)])

            @pl.when(i < _ITERS - 1)
            def _():
                start_gather(j + _NBUF, b)

        return carry

    lax.fori_loop(0, _ITERS, body, 0)


def _gather(table2, x2d):
    mesh = plsc.VectorSubcoreMesh(core_axis_name="c", subcore_axis_name="s")
    kfn = pl.kernel(
        _gather_body,
        out_type=jax.ShapeDtypeStruct((_PART, _DW), jnp.float32),
        mesh=mesh,
        scratch_types=[
            pltpu.VMEM((_CHUNKS, _C), jnp.int32),
            pltpu.VMEM((_NBUF, _C, _DW), jnp.float32),
            pltpu.SemaphoreType.DMA((_NBUF,)),
        ],
        compiler_params=pltpu.CompilerParams(use_tc_tiling_on_sc=False),
    )
    return kfn(table2, x2d)


_LSTEP = 2


def _assemble_body(g_ref, a_ref):
    for i in range(_LSTEP):
        w = lax.bitcast_convert_type(g_ref[i], jnp.uint32)
        lo = lax.bitcast_convert_type(
            (w & 0xFFFF).astype(jnp.uint16), jnp.bfloat16
        ).astype(jnp.float32)
        hi = lax.bitcast_convert_type(
            (w >> 16).astype(jnp.uint16), jnp.bfloat16
        ).astype(jnp.float32)
        loT = jnp.transpose(lo, (1, 0))
        hiT = jnp.transpose(hi, (1, 0))
        for m in range(4):
            a_ref[i, 0:D // 2, m * 4096:(m + 1) * 4096] = (
                loT[32 * m:32 * m + 32, :]
            )
            a_ref[i, D // 2:D, m * 4096:(m + 1) * 4096] = (
                hiT[32 * m:32 * m + 32, :]
            )


def _assemble_first(g3d, hist, batch, lsub):
    return pl.pallas_call(
        _assemble_body,
        grid=(lsub // _LSTEP,),
        in_specs=[
            pl.BlockSpec((_LSTEP, batch // 4, 2 * D), lambda l: (l, 0, 0))
        ],
        out_specs=pl.BlockSpec((_LSTEP, D, batch), lambda l: (l, 0, 0)),
        out_shape=jax.ShapeDtypeStruct((hist, D, batch), jnp.float32),
    )(g3d)


def _assemble_rest(g3d, acc, hist, batch, l0):
    def body(g_ref, _, a_ref):
        _assemble_body(g_ref, a_ref)

    lsub = hist - l0
    return pl.pallas_call(
        body,
        grid=(lsub // _LSTEP,),
        in_specs=[
            pl.BlockSpec((_LSTEP, batch // 4, 2 * D), lambda l: (l, 0, 0)),
            pl.BlockSpec(memory_space=pl.ANY),
        ],
        out_specs=pl.BlockSpec(
            (_LSTEP, D, batch), lambda l: (l + l0 // _LSTEP, 0, 0)
        ),
        out_shape=jax.ShapeDtypeStruct((hist, D, batch), jnp.float32),
        input_output_aliases={1: 0},
    )(g3d, acc)


def kernel(x, table, W, b):
    batch, hist = x.shape
    u = _transform(table.T, W, b.reshape(2, D // 2))

    xi = x.astype(jnp.int32).T.reshape(hist, 4, batch // 4)
    xi = jnp.transpose(xi, (0, 2, 1)).reshape(-1)
    qbits = _Q.bit_length() - 1
    idx = (
        (xi - jnp.bitwise_and(xi, _TBLK - 1))
        + (jnp.bitwise_and(xi, _Q - 1) << 2)
        + (jnp.bitwise_and(xi, _TBLK - 1) >> qbits)
    )
    x2d = idx.reshape(TOTAL // _C, _C)

    uw = u.reshape(4 * _UROWS, _DW)
    g = _gather(uw, x2d)
    a = _assemble_first(
        g.reshape(hist, batch // 4, 2 * D), hist, batch, hist
    )
    return jnp.transpose(a, (2, 0, 1))


# speedup vs baseline: 1.0318x; 1.0051x over previous
import jax
import jax.numpy as jnp
from jax import lax
from jax.experimental import pallas as pl
from jax.experimental.pallas import tpu as pltpu
from jax.experimental.pallas import tpu_sc as plsc

D = 64
NUM_ROWS = 1000000
TOTAL = 16384 * 50


_TBLK = 32768
_NBLKS = 31
_Q = _TBLK // 4
_UROWS = _NBLKS * _Q


def _transform_body(t_ref, w_ref, b_ref, o_ref):
    def dot_cols(sl, wrows):
        r = lax.dot_general(
            t_ref[:, sl], w_ref[wrows, :],
            (((0,), (1,)), ((), ())),
            preferred_element_type=jnp.float32,
        )
        return r

    def pack(sl):
        lo = dot_cols(sl, slice(0, D // 2)) + b_ref[0:1, :]
        hi = dot_cols(sl, slice(D // 2, D)) + b_ref[1:2, :]
        lo16 = lax.bitcast_convert_type(
            lo.astype(jnp.bfloat16), jnp.uint16
        ).astype(jnp.uint32)
        hi16 = lax.bitcast_convert_type(
            hi.astype(jnp.bfloat16), jnp.uint16
        ).astype(jnp.uint32)
        return lo16 | (hi16 << 16)

    quarters = [pack(pl.ds(k * _Q, _Q)) for k in range(4)]
    o_ref[...] = lax.bitcast_convert_type(
        jnp.concatenate(quarters, axis=1), jnp.float32
    )


def _transform(table_t, W, b2):
    return pl.pallas_call(
        _transform_body,
        grid=(_NBLKS,),
        in_specs=[
            pl.BlockSpec((D, _TBLK), lambda i: (0, i)),
            pl.BlockSpec((D, D), lambda i: (0, 0)),
            pl.BlockSpec((2, D // 2), lambda i: (0, 0)),
        ],
        out_specs=pl.BlockSpec((_Q, 2 * D), lambda i: (i, 0)),
        out_shape=jax.ShapeDtypeStruct((_UROWS, 2 * D), jnp.float32),
    )(table_t, W, b2)


_DW = D // 2
_C = 128
_NBUF = 4
_NSPLIT = 1
_info = plsc.get_sparse_core_info()
_NC, _NS = _info.num_cores, _info.num_subcores
_NW = _NC * _NS
_PART = TOTAL // _NSPLIT
_PER_W = _PART // _NW
_CHUNKS = _PER_W // _C
_ITERS = _CHUNKS // _NBUF


def _gather_body(table_hbm, idx_hbm, out_hbm, idx_v, bufs, gsems):
    wid = lax.axis_index("s") * _NC + lax.axis_index("c")
    chunk0 = wid * _CHUNKS
    row0 = wid * _PER_W

    pltpu.sync_copy(idx_hbm.at[pl.ds(chunk0, _CHUNKS)], idx_v)

    def start_gather(j, b):
        pltpu.async_copy(table_hbm.at[idx_v.at[j]], bufs.at[b], gsems.at[b])

    def wait_gather(j, b):
        pltpu.make_async_copy(
            table_hbm.at[idx_v.at[j]], bufs.at[b], gsems.at[b]
        ).wait()

    for b in range(_NBUF):
        start_gather(b, b)

    def body(i, carry):
        for b in range(_NBUF):
            j = i * _NBUF + b
            wait_gather(j, b)
            pltpu.sync_copy(bufs.at[b], out_hbm.at[pl.ds(row0 + j * _C, _C)])

            @pl.when(i < _ITERS - 1)
            def _():
                start_gather(j + _NBUF, b)

        return carry

    lax.fori_loop(0, _ITERS, body, 0)


def _gather(table2, x2d):
    mesh = plsc.VectorSubcoreMesh(core_axis_name="c", subcore_axis_name="s")
    kfn = pl.kernel(
        _gather_body,
        out_type=jax.ShapeDtypeStruct((_PART, _DW), jnp.float32),
        mesh=mesh,
        scratch_types=[
            pltpu.VMEM((_CHUNKS, _C), jnp.int32),
            pltpu.VMEM((_NBUF, _C, _DW), jnp.float32),
            pltpu.SemaphoreType.DMA((_NBUF,)),
        ],
        compiler_params=pltpu.CompilerParams(use_tc_tiling_on_sc=False),
    )
    return kfn(table2, x2d)


_LSTEP = 2


def _assemble_body(g_ref, a_ref):
    for i in range(_LSTEP):
        w = lax.bitcast_convert_type(g_ref[i], jnp.uint32)
        lo = lax.bitcast_convert_type(
            (w & 0xFFFF).astype(jnp.uint16), jnp.bfloat16
        ).astype(jnp.float32)
        hi = lax.bitcast_convert_type(
            (w >> 16).astype(jnp.uint16), jnp.bfloat16
        ).astype(jnp.float32)
        loT = jnp.transpose(lo, (1, 0))
        hiT = jnp.transpose(hi, (1, 0))
        for m in range(4):
            a_ref[i, 0:D // 2, m * 4096:(m + 1) * 4096] = (
                loT[32 * m:32 * m + 32, :]
            )
            a_ref[i, D // 2:D, m * 4096:(m + 1) * 4096] = (
                hiT[32 * m:32 * m + 32, :]
            )


def _assemble_first(g3d, hist, batch, lsub):
    return pl.pallas_call(
        _assemble_body,
        grid=(lsub // _LSTEP,),
        in_specs=[
            pl.BlockSpec((_LSTEP, batch // 4, 2 * D), lambda l: (l, 0, 0))
        ],
        out_specs=pl.BlockSpec((_LSTEP, D, batch), lambda l: (l, 0, 0)),
        out_shape=jax.ShapeDtypeStruct((hist, D, batch), jnp.float32),
    )(g3d)


def _assemble_rest(g3d, acc, hist, batch, l0):
    def body(g_ref, _, a_ref):
        _assemble_body(g_ref, a_ref)

    lsub = hist - l0
    return pl.pallas_call(
        body,
        grid=(lsub // _LSTEP,),
        in_specs=[
            pl.BlockSpec((_LSTEP, batch // 4, 2 * D), lambda l: (l, 0, 0)),
            pl.BlockSpec(memory_space=pl.ANY),
        ],
        out_specs=pl.BlockSpec(
            (_LSTEP, D, batch), lambda l: (l + l0 // _LSTEP, 0, 0)
        ),
        out_shape=jax.ShapeDtypeStruct((hist, D, batch), jnp.float32),
        input_output_aliases={1: 0},
    )(g3d, acc)


def kernel(x, table, W, b):
    batch, hist = x.shape
    u = _transform(table.T, W, b.reshape(2, D // 2))

    xi = x.astype(jnp.int32).T.reshape(hist, 4, batch // 4)
    xi = jnp.transpose(xi, (0, 2, 1)).reshape(-1)
    qbits = _Q.bit_length() - 1
    idx = (
        (xi - jnp.bitwise_and(xi, _TBLK - 1))
        + (jnp.bitwise_and(xi, _Q - 1) << 2)
        + (jnp.bitwise_and(xi, _TBLK - 1) >> qbits)
    )
    x2d = idx.reshape(TOTAL // _C, _C)

    uw = u.reshape(4 * _UROWS, _DW)
    g = _gather(uw, x2d)
    a = _assemble_first(
        g.reshape(hist, batch // 4, 2 * D), hist, batch, hist
    )
    return jnp.transpose(a, (2, 0, 1))


# speedup vs baseline: 1.0339x; 1.0020x over previous
import jax
import jax.numpy as jnp
from jax import lax
from jax.experimental import pallas as pl
from jax.experimental.pallas import tpu as pltpu
from jax.experimental.pallas import tpu_sc as plsc

D = 64
NUM_ROWS = 1000000
TOTAL = 16384 * 50


_TBLK = 32768
_NBLKS = 31
_Q = _TBLK // 4
_UROWS = _NBLKS * _Q


def _transform_body(t_ref, w_ref, b_ref, o_ref):
    def dot_cols(sl, wrows):
        r = lax.dot_general(
            t_ref[:, sl], w_ref[wrows, :],
            (((0,), (1,)), ((), ())),
            preferred_element_type=jnp.float32,
        )
        return r

    def pack(sl):
        lo = dot_cols(sl, slice(0, D // 2)) + b_ref[0:1, :]
        hi = dot_cols(sl, slice(D // 2, D)) + b_ref[1:2, :]
        lo16 = lax.bitcast_convert_type(
            lo.astype(jnp.bfloat16), jnp.uint16
        ).astype(jnp.uint32)
        hi16 = lax.bitcast_convert_type(
            hi.astype(jnp.bfloat16), jnp.uint16
        ).astype(jnp.uint32)
        return lo16 | (hi16 << 16)

    quarters = [pack(pl.ds(k * _Q, _Q)) for k in range(4)]
    o_ref[...] = lax.bitcast_convert_type(
        jnp.concatenate(quarters, axis=1), jnp.float32
    )


def _transform(table_t, W, b2):
    return pl.pallas_call(
        _transform_body,
        grid=(_NBLKS,),
        in_specs=[
            pl.BlockSpec((D, _TBLK), lambda i: (0, i)),
            pl.BlockSpec((D, D), lambda i: (0, 0)),
            pl.BlockSpec((2, D // 2), lambda i: (0, 0)),
        ],
        out_specs=pl.BlockSpec((_Q, 2 * D), lambda i: (i, 0)),
        out_shape=jax.ShapeDtypeStruct((_UROWS, 2 * D), jnp.float32),
    )(table_t, W, b2)


_DW = D // 2
_C = 256
_NBUF = 4
_NSPLIT = 1
_info = plsc.get_sparse_core_info()
_NC, _NS = _info.num_cores, _info.num_subcores
_NW = _NC * _NS
_PART = TOTAL // _NSPLIT
_PER_W = _PART // _NW
_CHUNKS = _PER_W // _C
_ITERS = _CHUNKS // _NBUF


def _gather_body(table_hbm, idx_hbm, out_hbm, idx_v, bufs, gsems):
    wid = lax.axis_index("s") * _NC + lax.axis_index("c")
    chunk0 = wid * _CHUNKS
    row0 = wid * _PER_W

    pltpu.sync_copy(idx_hbm.at[pl.ds(chunk0, _CHUNKS)], idx_v)

    def start_gather(j, b):
        pltpu.async_copy(table_hbm.at[idx_v.at[j]], bufs.at[b], gsems.at[b])

    def wait_gather(j, b):
        pltpu.make_async_copy(
            table_hbm.at[idx_v.at[j]], bufs.at[b], gsems.at[b]
        ).wait()

    for b in range(_NBUF):
        start_gather(b, b)

    def body(i, carry):
        for b in range(_NBUF):
            j = i * _NBUF + b
            wait_gather(j, b)
            pltpu.sync_copy(bufs.at[b], out_hbm.at[pl.ds(row0 + j * _C, _C)])

            @pl.when(i < _ITERS - 1)
            def _():
                start_gather(j + _NBUF, b)

        return carry

    lax.fori_loop(0, _ITERS, body, 0)


def _gather(table2, x2d):
    mesh = plsc.VectorSubcoreMesh(core_axis_name="c", subcore_axis_name="s")
    kfn = pl.kernel(
        _gather_body,
        out_type=jax.ShapeDtypeStruct((_PART, _DW), jnp.float32),
        mesh=mesh,
        scratch_types=[
            pltpu.VMEM((_CHUNKS, _C), jnp.int32),
            pltpu.VMEM((_NBUF, _C, _DW), jnp.float32),
            pltpu.SemaphoreType.DMA((_NBUF,)),
        ],
        compiler_params=pltpu.CompilerParams(use_tc_tiling_on_sc=False),
    )
    return kfn(table2, x2d)


_LSTEP = 2


def _assemble_body(g_ref, a_ref):
    for i in range(_LSTEP):
        w = lax.bitcast_convert_type(g_ref[i], jnp.uint32)
        lo = lax.bitcast_convert_type(
            (w & 0xFFFF).astype(jnp.uint16), jnp.bfloat16
        ).astype(jnp.float32)
        hi = lax.bitcast_convert_type(
            (w >> 16).astype(jnp.uint16), jnp.bfloat16
        ).astype(jnp.float32)
        loT = jnp.transpose(lo, (1, 0))
        hiT = jnp.transpose(hi, (1, 0))
        for m in range(4):
            a_ref[i, 0:D // 2, m * 4096:(m + 1) * 4096] = (
                loT[32 * m:32 * m + 32, :]
            )
            a_ref[i, D // 2:D, m * 4096:(m + 1) * 4096] = (
                hiT[32 * m:32 * m + 32, :]
            )


def _assemble_first(g3d, hist, batch, lsub):
    return pl.pallas_call(
        _assemble_body,
        grid=(lsub // _LSTEP,),
        in_specs=[
            pl.BlockSpec((_LSTEP, batch // 4, 2 * D), lambda l: (l, 0, 0))
        ],
        out_specs=pl.BlockSpec((_LSTEP, D, batch), lambda l: (l, 0, 0)),
        out_shape=jax.ShapeDtypeStruct((hist, D, batch), jnp.float32),
    )(g3d)


def _assemble_rest(g3d, acc, hist, batch, l0):
    def body(g_ref, _, a_ref):
        _assemble_body(g_ref, a_ref)

    lsub = hist - l0
    return pl.pallas_call(
        body,
        grid=(lsub // _LSTEP,),
        in_specs=[
            pl.BlockSpec((_LSTEP, batch // 4, 2 * D), lambda l: (l, 0, 0)),
            pl.BlockSpec(memory_space=pl.ANY),
        ],
        out_specs=pl.BlockSpec(
            (_LSTEP, D, batch), lambda l: (l + l0 // _LSTEP, 0, 0)
        ),
        out_shape=jax.ShapeDtypeStruct((hist, D, batch), jnp.float32),
        input_output_aliases={1: 0},
    )(g3d, acc)


def kernel(x, table, W, b):
    batch, hist = x.shape
    u = _transform(table.T, W, b.reshape(2, D // 2))

    xi = x.astype(jnp.int32).T.reshape(hist, 4, batch // 4)
    xi = jnp.transpose(xi, (0, 2, 1)).reshape(-1)
    qbits = _Q.bit_length() - 1
    idx = (
        (xi - jnp.bitwise_and(xi, _TBLK - 1))
        + (jnp.bitwise_and(xi, _Q - 1) << 2)
        + (jnp.bitwise_and(xi, _TBLK - 1) >> qbits)
    )
    x2d = idx.reshape(TOTAL // _C, _C)

    uw = u.reshape(4 * _UROWS, _DW)
    g = _gather(uw, x2d)
    a = _assemble_first(
        g.reshape(hist, batch // 4, 2 * D), hist, batch, hist
    )
    return jnp.transpose(a, (2, 0, 1))
